# baseline (speedup 1.0000x reference)
import functools

import jax
import jax.numpy as jnp
from jax import lax
from jax.experimental import pallas as pl
from jax.experimental.pallas import tpu as pltpu
from jax.experimental.pallas import tpu_sc as plsc

_NUM_CORES = 2
_NUM_SUBCORES = 16
_NUM_WORKERS = _NUM_CORES * _NUM_SUBCORES


@functools.lru_cache(maxsize=None)
def _make_copy_kernel(seq_len: int, dim: int):
    rows_per_w = seq_len // _NUM_WORKERS
    mesh = plsc.VectorSubcoreMesh(core_axis_name="c", subcore_axis_name="s")

    @functools.partial(
        pl.kernel,
        mesh=mesh,
        out_type=jax.ShapeDtypeStruct((seq_len, dim), jnp.float32),
        scratch_types=[
            pltpu.SemaphoreType.DMA,
        ],
    )
    def k(emb_hbm, out_hbm, sem):
        wid = lax.axis_index("s") * _NUM_CORES + lax.axis_index("c")
        base = wid * rows_per_w
        pltpu.async_copy(
            emb_hbm.at[pl.ds(base, rows_per_w)],
            out_hbm.at[pl.ds(base, rows_per_w)],
            sem,
        ).wait()

    return k


def kernel(x, emb_weight):
    seq_len = x.shape[1]
    dim = emb_weight.shape[1]
    return _make_copy_kernel(seq_len, dim)(emb_weight)


# speedup vs baseline: 24.2026x; 24.2026x over previous
import functools

import jax
import jax.numpy as jnp
from jax import lax
from jax.experimental import pallas as pl
from jax.experimental.pallas import tpu as pltpu
from jax.experimental.pallas import tpu_sc as plsc

_NUM_CORES = 2
_NUM_SUBCORES = 16
_NUM_WORKERS = _NUM_CORES * _NUM_SUBCORES
_CHUNK_ROWS = 32


@functools.lru_cache(maxsize=None)
def _make_copy_kernel(seq_len: int, dim: int):
    rows_per_w = seq_len // _NUM_WORKERS
    chunk = min(rows_per_w, _CHUNK_ROWS)
    nchunk = rows_per_w // chunk
    mesh = plsc.VectorSubcoreMesh(core_axis_name="c", subcore_axis_name="s")

    @functools.partial(
        pl.kernel,
        mesh=mesh,
        out_type=jax.ShapeDtypeStruct((seq_len, dim), jnp.float32),
        scratch_types=[
            pltpu.VMEM((chunk, dim), jnp.float32),
            pltpu.VMEM((chunk, dim), jnp.float32),
            pltpu.SemaphoreType.DMA,
            pltpu.SemaphoreType.DMA,
            pltpu.SemaphoreType.DMA,
            pltpu.SemaphoreType.DMA,
        ],
    )
    def k(emb_hbm, out_hbm, buf0, buf1, rsem0, rsem1, wsem0, wsem1):
        bufs = (buf0, buf1)
        rsems = (rsem0, rsem1)
        wsems = (wsem0, wsem1)
        wid = lax.axis_index("s") * _NUM_CORES + lax.axis_index("c")
        base = wid * rows_per_w

        def read(c):
            b = c % 2
            return pltpu.async_copy(
                emb_hbm.at[pl.ds(base + c * chunk, chunk)], bufs[b], rsems[b])

        def write(c):
            b = c % 2
            return pltpu.async_copy(
                bufs[b], out_hbm.at[pl.ds(base + c * chunk, chunk)], wsems[b])

        reads = {0: read(0)}
        writes = {}
        for c in range(nchunk):
            if c + 1 < nchunk:
                if c - 1 >= 0:
                    writes.pop(c - 1).wait()
                reads[c + 1] = read(c + 1)
            reads.pop(c).wait()
            writes[c] = write(c)
        for w in writes.values():
            w.wait()

    return k


def kernel(x, emb_weight):
    seq_len = x.shape[1]
    dim = emb_weight.shape[1]
    return _make_copy_kernel(seq_len, dim)(emb_weight)


# speedup vs baseline: 41.7006x; 1.7230x over previous
import functools

import jax
import jax.numpy as jnp
from jax.experimental import pallas as pl
from jax.experimental.pallas import tpu as pltpu

_BLOCK_ROWS = 512


def _copy_body(in_ref, out_ref):
    out_ref[...] = in_ref[...]


@functools.lru_cache(maxsize=None)
def _make_copy_kernel(seq_len: int, dim: int):
    grid = seq_len // _BLOCK_ROWS
    return pl.pallas_call(
        _copy_body,
        grid=(grid,),
        in_specs=[pl.BlockSpec((_BLOCK_ROWS, dim), lambda i: (i, 0))],
        out_specs=pl.BlockSpec((_BLOCK_ROWS, dim), lambda i: (i, 0)),
        out_shape=jax.ShapeDtypeStruct((seq_len, dim), jnp.float32),
    )


def kernel(x, emb_weight):
    seq_len = x.shape[1]
    dim = emb_weight.shape[1]
    return _make_copy_kernel(seq_len, dim)(emb_weight)
